# baseline (speedup 1.0000x reference)
import functools

import jax
import jax.numpy as jnp
from jax.experimental import pallas as pl
from jax.experimental.pallas import tpu as pltpu

_TEMPERATURE = 0.2
_K = 5
_NEG = -1e30


def _select_emit(sims, o_ref):
    def pair(arr):
        half = arr.shape[1] // 2
        return (jnp.maximum(arr[:, :half], arr[:, half:]),
                jnp.minimum(arr[:, :half], arr[:, half:]))

    def top_vals(arr, kk):
        vs = [jnp.max(arr, axis=1, keepdims=True)]
        for _ in range(kk - 1):
            masked = jnp.where(arr < vs[-1], arr, _NEG)
            vs.append(jnp.max(masked, axis=1, keepdims=True))
        return vs

    def cands(arr, kk):
        w = arr.shape[1]
        if kk == 1:
            return [jnp.max(arr, axis=1, keepdims=True)]
        if (kk > 2 and w <= 512) or (kk <= 2 and w <= 1024):
            return top_vals(arr, kk)
        hi, lo = pair(arr)
        return cands(hi, kk) + cands(lo, kk // 2)

    vals = top_vals(jnp.concatenate(cands(sims, _K), axis=1), _K)
    es = [jnp.exp((v - vals[0]) / _TEMPERATURE) for v in vals]
    denom = functools.reduce(jnp.add, es)
    c1 = 1.4426950408889634 / _TEMPERATURE
    s = -vals[0] * c1 - jnp.log(denom) * 1.4426950408889634
    w = jnp.exp2(sims * c1 + s)
    o_ref[...] = jnp.where(sims >= vals[_K - 1], w, 0.0)


def _body(x_ref, p_ref, o_ref, s_ref, *, nt):
    i = pl.program_id(0)

    @pl.when(i < nt)
    def _matmul():
        x = x_ref[...]
        nrm = jnp.sqrt(jnp.sum(x * x, axis=1, keepdims=True))
        xn = x / jnp.maximum(nrm, 1e-12)
        s_ref[i % 2] = jax.lax.dot_general(
            xn, p_ref[...], (((1,), (1,)), ((), ())),
            preferred_element_type=jnp.float32,
        )

    @pl.when(i > 0)
    def _select():
        _select_emit(s_ref[(i - 1) % 2], o_ref)


def kernel(x, prototypes, k):
    del k
    if x.ndim == 1:
        x = x[None, :]
    b, d = x.shape
    n = prototypes.shape[0]
    bt = 256
    nt = b // bt
    return pl.pallas_call(
        functools.partial(_body, nt=nt),
        grid=(nt + 1,),
        in_specs=[
            pl.BlockSpec((bt, d), lambda i: (jnp.minimum(i, nt - 1), 0)),
            pl.BlockSpec((n, d), lambda i: (0, 0)),
        ],
        out_specs=pl.BlockSpec((bt, n), lambda i: (jnp.maximum(i - 1, 0), 0)),
        out_shape=jax.ShapeDtypeStruct((b, n), jnp.float32),
        scratch_shapes=[pltpu.VMEM((2, bt, n), jnp.float32)],
    )(x, prototypes)


# speedup vs baseline: 1.0635x; 1.0635x over previous
import functools

import jax
import jax.numpy as jnp
from jax.experimental import pallas as pl

_TEMPERATURE = 0.2
_K = 5
_NEG = -1e30


def _body(x_ref, p_ref, o_ref):
    p = p_ref[...]
    h = x_ref.shape[0] // 2

    def mm(x):
        nrm = jnp.sqrt(jnp.sum(x * x, axis=1, keepdims=True))
        xn = x / jnp.maximum(nrm, 1e-12)
        return jax.lax.dot_general(
            xn, p, (((1,), (1,)), ((), ())),
            preferred_element_type=jnp.float32,
        )

    sims_halves = [mm(x_ref[:h]), mm(x_ref[h:])]
    for hh_i, sims in enumerate(sims_halves):
        _sel(sims, o_ref, hh_i * h)


def _sel(sims, o_ref, row0):
    def pair(arr):
        half = arr.shape[1] // 2
        return (jnp.maximum(arr[:, :half], arr[:, half:]),
                jnp.minimum(arr[:, :half], arr[:, half:]))

    def top_vals(arr, kk):
        vs = [jnp.max(arr, axis=1, keepdims=True)]
        for _ in range(kk - 1):
            masked = jnp.where(arr < vs[-1], arr, _NEG)
            vs.append(jnp.max(masked, axis=1, keepdims=True))
        return vs

    def top1(arr):
        return [jnp.max(arr, axis=1, keepdims=True)]

    hi, lo = pair(sims)
    hh, hl = pair(hi)
    lh, ll = pair(lo)
    hhh, hhl = pair(hh)
    hlh, hll = pair(hl)
    lhh, lhl = pair(lh)
    cands = (top_vals(hhh, 5) + top_vals(hhl, 2) + top_vals(hlh, 2)
             + top1(hll) + top_vals(lhh, 2) + top1(lhl) + top1(ll))
    vals = top_vals(jnp.concatenate(cands, axis=1), _K)
    es = [jnp.exp((v - vals[0]) / _TEMPERATURE) for v in vals]
    denom = functools.reduce(jnp.add, es)
    c1 = 1.4426950408889634 / _TEMPERATURE
    s = -vals[0] * c1 - jnp.log(denom) * 1.4426950408889634
    w = jnp.exp2(sims * c1 + s)
    o_ref[row0:row0 + sims.shape[0], :] = jnp.where(
        sims >= vals[_K - 1], w, 0.0)


def kernel(x, prototypes, k):
    del k
    if x.ndim == 1:
        x = x[None, :]
    b, d = x.shape
    n = prototypes.shape[0]
    bt = 256
    grid = (b // bt,)
    return pl.pallas_call(
        _body,
        grid=grid,
        in_specs=[
            pl.BlockSpec((bt, d), lambda i: (i, 0)),
            pl.BlockSpec((n, d), lambda i: (0, 0)),
        ],
        out_specs=pl.BlockSpec((bt, n), lambda i: (i, 0)),
        out_shape=jax.ShapeDtypeStruct((b, n), jnp.float32),
    )(x, prototypes)


# speedup vs baseline: 1.0715x; 1.0075x over previous
import functools

import jax
import jax.numpy as jnp
from jax.experimental import pallas as pl

_TEMPERATURE = 0.2
_K = 5
_NEG = -1e30


def _body(x_ref, p_ref, o_ref):
    x = x_ref[...]
    nrm = jnp.sqrt(jnp.sum(x * x, axis=1, keepdims=True))
    xn = x / jnp.maximum(nrm, 1e-12)
    sims = jax.lax.dot_general(
        xn, p_ref[...], (((1,), (1,)), ((), ())),
        preferred_element_type=jnp.float32,
    )
    def pair(arr):
        half = arr.shape[1] // 2
        return (jnp.maximum(arr[:, :half], arr[:, half:]),
                jnp.minimum(arr[:, :half], arr[:, half:]))

    def top_vals(arr, kk):
        vs = [jnp.max(arr, axis=1, keepdims=True)]
        for _ in range(kk - 1):
            masked = jnp.where(arr < vs[-1], arr, _NEG)
            vs.append(jnp.max(masked, axis=1, keepdims=True))
        return vs

    def top1(arr):
        return [jnp.max(arr, axis=1, keepdims=True)]

    hi, lo = pair(sims)
    hh, hl = pair(hi)
    lh, ll = pair(lo)
    hhh, hhl = pair(hh)
    hlh, hll = pair(hl)
    lhh, lhl = pair(lh)
    cands = (top_vals(hhh, 5) + top_vals(hhl, 2) + top_vals(hlh, 2)
             + top1(hll) + top_vals(lhh, 2) + top1(lhl) + top1(ll))
    vals = top_vals(jnp.concatenate(cands, axis=1), _K)
    es = [jnp.exp((v - vals[0]) / _TEMPERATURE) for v in vals]
    denom = functools.reduce(jnp.add, es)
    c1 = 1.4426950408889634 / _TEMPERATURE
    s = -vals[0] * c1 - jnp.log(denom) * 1.4426950408889634
    w = jnp.exp2(sims * c1 + s)
    o_ref[...] = jnp.where(sims >= vals[_K - 1], w, 0.0)


def kernel(x, prototypes, k):
    del k
    if x.ndim == 1:
        x = x[None, :]
    b, d = x.shape
    n = prototypes.shape[0]
    bt = 256
    grid = (b // bt,)
    return pl.pallas_call(
        _body,
        grid=grid,
        in_specs=[
            pl.BlockSpec((bt, d), lambda i: (i, 0)),
            pl.BlockSpec((n, d), lambda i: (0, 0)),
        ],
        out_specs=pl.BlockSpec((bt, n), lambda i: (i, 0)),
        out_shape=jax.ShapeDtypeStruct((b, n), jnp.float32),
    )(x, prototypes)
